# Initial kernel scaffold; baseline (speedup 1.0000x reference)
import jax
import jax.numpy as jnp
from jax.experimental import pallas as pl


def kernel(x, edge_index, edge_pairs, W0, b0, g0, be0, W1, b1, g1, be1, Wp, d1W, d1b, d1g, d1be, d2W, d2b, d2g, d2be, d3W, d3b):
    raise NotImplementedError("write your pallas kernel here")


# speedup vs baseline: 3.0766x; 3.0766x over previous
import jax
import jax.numpy as jnp
from jax import lax
from jax.experimental import pallas as pl
from jax.experimental.pallas import tpu as pltpu
from jax.experimental.pallas import tpu_sc as plsc

N = 10000
E = 320000
P = 320000
D = 128
H = 128
DEC = 256
EPS = 1e-5

NC, NS = 2, 16
NW = NC * NS
CHUNK = 128
CPT = 79
EPAD = NW * CPT * CHUNK
NP = N + 8
ZR = 640
ZLAST = NP - ZR

BP = 512
GD = P // BP

_mesh_cache = []


def _mesh():
    if not _mesh_cache:
        _mesh_cache.append(plsc.VectorSubcoreMesh(
            core_axis_name="c", subcore_axis_name="s",
            num_cores=NC, num_subcores=NS))
    return _mesh_cache[0]


IPT = CPT * CHUNK
VSTEPS = IPT // 16


def _deg_body(ones_hbm, dst_hbm, out_hbm, idx_v, ones_v, acc_sh, sem):
    c = lax.axis_index("c")
    s = lax.axis_index("s")
    wid = s * NC + c
    zoff = jnp.minimum(s * ZR, ZLAST)

    pltpu.sync_copy(ones_hbm.at[pl.ds(0, 16)], ones_v)
    pltpu.sync_copy(ones_hbm.at[pl.ds(zoff, ZR)], acc_sh.at[pl.ds(zoff, ZR)])
    pltpu.sync_copy(dst_hbm.at[wid], idx_v)
    plsc.subcore_barrier()

    def step(t, _):
        dvec = idx_v[0, pl.ds(16 * t, 16)]
        pltpu.sync_copy(ones_v, acc_sh.at[dvec], add=True)
        return 0
    lax.fori_loop(0, VSTEPS, step, 0)

    plsc.subcore_barrier()
    pltpu.sync_copy(acc_sh.at[pl.ds(zoff, ZR)], out_hbm.at[c, pl.ds(zoff, ZR)])


def _sc_deg(ones, dst3d):
    return pl.kernel(
        _deg_body,
        out_type=jax.ShapeDtypeStruct((NC, NP, H), jnp.float32),
        mesh=_mesh(),
        scratch_types=[
            pltpu.VMEM((1, IPT), jnp.int32),
            pltpu.VMEM((16, H), jnp.float32),
            pltpu.VMEM_SHARED((NP, H), jnp.float32),
            pltpu.SemaphoreType.DMA,
        ],
    )(ones, dst3d)


def _conv_body(tbl_hbm, src_hbm, dst_hbm, out_hbm,
               sidx, didx, rows, acc_sh, sem):
    c = lax.axis_index("c")
    s = lax.axis_index("s")
    wid = s * NC + c
    zoff = jnp.minimum(s * ZR, ZLAST)

    pltpu.sync_copy(tbl_hbm.at[pl.ds(zoff, ZR)], acc_sh.at[pl.ds(zoff, ZR)])
    plsc.subcore_barrier()
    pltpu.sync_copy(src_hbm.at[wid], sidx)
    pltpu.sync_copy(dst_hbm.at[wid], didx)

    def step(t, _):
        svec = sidx[0, pl.ds(16 * t, 16)]
        dvec = didx[0, pl.ds(16 * t, 16)]
        pltpu.async_copy(tbl_hbm.at[svec], rows, sem).wait()
        pltpu.sync_copy(rows, acc_sh.at[dvec], add=True)
        return 0
    lax.fori_loop(0, VSTEPS, step, 0)

    plsc.subcore_barrier()
    pltpu.sync_copy(acc_sh.at[pl.ds(zoff, ZR)], out_hbm.at[c, pl.ds(zoff, ZR)])


def _sc_conv(tbl, src3d, dst3d):
    return pl.kernel(
        _conv_body,
        out_type=jax.ShapeDtypeStruct((NC, NP, H), jnp.float32),
        mesh=_mesh(),
        scratch_types=[
            pltpu.VMEM((1, IPT), jnp.int32),
            pltpu.VMEM((1, IPT), jnp.int32),
            pltpu.VMEM((16, H), jnp.float32),
            pltpu.VMEM_SHARED((NP, H), jnp.float32),
            pltpu.SemaphoreType.DMA,
        ],
    )(tbl, src3d, dst3d)


def _pair_body(z_hbm, s_hbm, d_hbm, u_hbm, v_hbm,
               sidx, didx, ub, vb, sem1, sem2):
    c = lax.axis_index("c")
    s = lax.axis_index("s")
    wid = s * NC + c
    pltpu.sync_copy(s_hbm.at[wid], sidx)
    pltpu.sync_copy(d_hbm.at[wid], didx)

    def step(t, _):
        base = wid * IPT + 16 * t
        svec = sidx[0, pl.ds(16 * t, 16)]
        dvec = didx[0, pl.ds(16 * t, 16)]
        cp1 = pltpu.async_copy(z_hbm.at[svec], ub, sem1)
        cp2 = pltpu.async_copy(z_hbm.at[dvec], vb, sem2)
        cp1.wait()
        cp2.wait()
        pltpu.sync_copy(ub, u_hbm.at[pl.ds(base, 16)])
        pltpu.sync_copy(vb, v_hbm.at[pl.ds(base, 16)])
        return 0
    lax.fori_loop(0, VSTEPS, step, 0)


def _sc_pair(z, s3d, d3d):
    return pl.kernel(
        _pair_body,
        out_type=(jax.ShapeDtypeStruct((EPAD, H), jnp.float32),
                  jax.ShapeDtypeStruct((EPAD, H), jnp.float32)),
        mesh=_mesh(),
        scratch_types=[
            pltpu.VMEM((1, IPT), jnp.int32),
            pltpu.VMEM((1, IPT), jnp.int32),
            pltpu.VMEM((16, H), jnp.float32),
            pltpu.VMEM((16, H), jnp.float32),
            pltpu.SemaphoreType.DMA,
            pltpu.SemaphoreType.DMA,
        ],
    )(z, s3d, d3d)


def _prep_body(degp_ref, x_ref, W0_ref, Wp_ref, hs0_ref, xp_ref, dinv_ref):
    deg = degp_ref[0, :N, 0:1] + degp_ref[1, :N, 0:1] - 1.0
    dinv = lax.rsqrt(deg)
    x = x_ref[...]
    hs0_ref[...] = jnp.dot(x, W0_ref[...],
                           preferred_element_type=jnp.float32) * dinv
    xp_ref[...] = jnp.dot(x, Wp_ref[...], preferred_element_type=jnp.float32)
    dinv_ref[...] = dinv


def _tc_prep(degp, x, W0, Wp):
    return pl.pallas_call(
        _prep_body,
        out_shape=(jax.ShapeDtypeStruct((N, H), jnp.float32),
                   jax.ShapeDtypeStruct((N, H), jnp.float32),
                   jax.ShapeDtypeStruct((N, 1), jnp.float32)),
    )(degp, x, W0, Wp)


def _enc1_body(S_ref, hs_ref, dinv_ref, xp_ref, b_ref, g_ref, be_ref, W1_ref,
               h_ref, hs1_ref):
    S = S_ref[0, :N] + S_ref[1, :N] - hs_ref[...]
    dinv = dinv_ref[...]
    pre = S * dinv + b_ref[...]
    mu = jnp.mean(pre, axis=0, keepdims=True)
    var = jnp.mean((pre - mu) ** 2, axis=0, keepdims=True)
    hb = g_ref[...] * (pre - mu) * lax.rsqrt(var + EPS) + be_ref[...]
    h = jnp.maximum(hb, 0.0) + xp_ref[...]
    h_ref[...] = h
    hs1_ref[...] = jnp.dot(h, W1_ref[...],
                           preferred_element_type=jnp.float32) * dinv


def _tc_enc1(S0, hs0, dinv, xp, b0, g0, be0, W1):
    return pl.pallas_call(
        _enc1_body,
        out_shape=(jax.ShapeDtypeStruct((N, H), jnp.float32),
                   jax.ShapeDtypeStruct((N, H), jnp.float32)),
    )(S0, hs0, dinv, xp, b0, g0, be0, W1)


def _enc2_body(S_ref, hs1_ref, dinv_ref, h_ref, b_ref, g_ref, be_ref, z_ref):
    S = S_ref[0, :N] + S_ref[1, :N] - hs1_ref[...]
    pre = S * dinv_ref[...] + b_ref[...]
    mu = jnp.mean(pre, axis=0, keepdims=True)
    var = jnp.mean((pre - mu) ** 2, axis=0, keepdims=True)
    hb = g_ref[...] * (pre - mu) * lax.rsqrt(var + EPS) + be_ref[...]
    z_ref[...] = jnp.maximum(hb, 0.0) + h_ref[...]


def _tc_enc2(S1, hs1, dinv, h, b1, g1, be1):
    return pl.pallas_call(
        _enc2_body,
        out_shape=jax.ShapeDtypeStruct((N, H), jnp.float32),
    )(S1, hs1, dinv, h, b1, g1, be1)


def _dec1_body(u_ref, v_ref, W_ref, b_ref, o_ref, st_ref):
    u = u_ref[...]
    v = v_ref[...]
    o = (jnp.dot(u, W_ref[0], preferred_element_type=jnp.float32)
         + jnp.dot(v, W_ref[1], preferred_element_type=jnp.float32)
         + jnp.dot(u * v, W_ref[2], preferred_element_type=jnp.float32)
         + jnp.dot(jnp.abs(u - v), W_ref[3], preferred_element_type=jnp.float32)
         + b_ref[...])
    o_ref[...] = o
    st = jnp.concatenate(
        [jnp.sum(o, axis=0, keepdims=True),
         jnp.sum(o * o, axis=0, keepdims=True),
         jnp.zeros((6, o.shape[1]), jnp.float32)], axis=0)

    @pl.when(pl.program_id(0) == 0)
    def _():
        st_ref[...] = st

    @pl.when(pl.program_id(0) > 0)
    def _():
        st_ref[...] += st


def _tc_dec1(u, v, W4, b):
    return pl.pallas_call(
        _dec1_body,
        grid=(GD,),
        in_specs=[
            pl.BlockSpec((BP, H), lambda i: (i, 0)),
            pl.BlockSpec((BP, H), lambda i: (i, 0)),
            pl.BlockSpec((4, H, DEC), lambda i: (0, 0, 0)),
            pl.BlockSpec((1, DEC), lambda i: (0, 0)),
        ],
        out_specs=(pl.BlockSpec((BP, DEC), lambda i: (i, 0)),
                   pl.BlockSpec((8, DEC), lambda i: (0, 0))),
        out_shape=(jax.ShapeDtypeStruct((P, DEC), jnp.float32),
                   jax.ShapeDtypeStruct((8, DEC), jnp.float32)),
    )(u, v, W4, b)


def _dec2_body(o1_ref, st_ref, g_ref, be_ref, W_ref, b_ref, o2_ref, st2_ref):
    mu = st_ref[0:1] * (1.0 / P)
    var = st_ref[1:2] * (1.0 / P) - mu * mu
    a = g_ref[...] * lax.rsqrt(var + EPS)
    y = jnp.maximum(a * (o1_ref[...] - mu) + be_ref[...], 0.0)
    o2 = jnp.dot(y, W_ref[...], preferred_element_type=jnp.float32) + b_ref[...]
    o2_ref[...] = o2
    st = jnp.concatenate(
        [jnp.sum(o2, axis=0, keepdims=True),
         jnp.sum(o2 * o2, axis=0, keepdims=True),
         jnp.zeros((6, o2.shape[1]), jnp.float32)], axis=0)

    @pl.when(pl.program_id(0) == 0)
    def _():
        st2_ref[...] = st

    @pl.when(pl.program_id(0) > 0)
    def _():
        st2_ref[...] += st


def _tc_dec2(o1, st1, g, be, W, b):
    return pl.pallas_call(
        _dec2_body,
        grid=(GD,),
        in_specs=[
            pl.BlockSpec((BP, DEC), lambda i: (i, 0)),
            pl.BlockSpec((8, DEC), lambda i: (0, 0)),
            pl.BlockSpec((1, DEC), lambda i: (0, 0)),
            pl.BlockSpec((1, DEC), lambda i: (0, 0)),
            pl.BlockSpec((DEC, DEC // 2), lambda i: (0, 0)),
            pl.BlockSpec((1, DEC // 2), lambda i: (0, 0)),
        ],
        out_specs=(pl.BlockSpec((BP, DEC // 2), lambda i: (i, 0)),
                   pl.BlockSpec((8, DEC // 2), lambda i: (0, 0))),
        out_shape=(jax.ShapeDtypeStruct((P, DEC // 2), jnp.float32),
                   jax.ShapeDtypeStruct((8, DEC // 2), jnp.float32)),
    )(o1, st1, g, be, W, b)


def _dec3_body(o2_ref, st_ref, g_ref, be_ref, w_ref, b_ref, out_ref):
    mu = st_ref[0:1] * (1.0 / P)
    var = st_ref[1:2] * (1.0 / P) - mu * mu
    a = g_ref[...] * lax.rsqrt(var + EPS)
    y = jnp.maximum(a * (o2_ref[...] - mu) + be_ref[...], 0.0)
    out_ref[...] = jnp.sum(y * w_ref[...], axis=1) + b_ref[0, 0]


def _tc_dec3(o2, st2, g, be, w_row, b):
    return pl.pallas_call(
        _dec3_body,
        grid=(GD,),
        in_specs=[
            pl.BlockSpec((BP, DEC // 2), lambda i: (i, 0)),
            pl.BlockSpec((8, DEC // 2), lambda i: (0, 0)),
            pl.BlockSpec((1, DEC // 2), lambda i: (0, 0)),
            pl.BlockSpec((1, DEC // 2), lambda i: (0, 0)),
            pl.BlockSpec((1, DEC // 2), lambda i: (0, 0)),
            pl.BlockSpec((1, 1), lambda i: (0, 0)),
        ],
        out_specs=pl.BlockSpec((BP,), lambda i: (i,)),
        out_shape=jax.ShapeDtypeStruct((P,), jnp.float32),
    )(o2, st2, g, be, w_row, b)


def _pad_idx(a, n_total):
    pad = jnp.full((n_total - a.shape[0],), N, dtype=a.dtype)
    return jnp.concatenate([a, pad]).reshape(NW, 1, CPT * CHUNK)


def _pad_rows(t):
    return jnp.pad(t, ((0, NP - t.shape[0]), (0, 0)))


def kernel(x, edge_index, edge_pairs, W0, b0, g0, be0, W1, b1, g1, be1, Wp,
           d1W, d1b, d1g, d1be, d2W, d2b, d2g, d2be, d3W, d3b):
    src = _pad_idx(edge_index[0], EPAD)
    dst = _pad_idx(edge_index[1], EPAD)
    ps = _pad_idx(edge_pairs[:, 0], EPAD)
    pd = _pad_idx(edge_pairs[:, 1], EPAD)

    degp = _sc_deg(jnp.ones((NP, H), jnp.float32), dst)
    hs0, xp, dinv = _tc_prep(degp, x, W0, Wp)
    S0 = _sc_conv(_pad_rows(hs0), src, dst)
    h, hs1 = _tc_enc1(S0, hs0, dinv, xp, b0.reshape(1, H), g0.reshape(1, H),
                      be0.reshape(1, H), W1)
    S1 = _sc_conv(_pad_rows(hs1), src, dst)
    z = _tc_enc2(S1, hs1, dinv, h, b1.reshape(1, H), g1.reshape(1, H),
                 be1.reshape(1, H))
    u, v = _sc_pair(_pad_rows(z), ps, pd)
    o1, st1 = _tc_dec1(u, v, d1W.reshape(4, H, DEC), d1b.reshape(1, DEC))
    o2, st2 = _tc_dec2(o1, st1, d1g.reshape(1, DEC), d1be.reshape(1, DEC),
                       d2W, d2b.reshape(1, DEC // 2))
    out = _tc_dec3(o2, st2, d2g.reshape(1, DEC // 2), d2be.reshape(1, DEC // 2),
                   d3W.reshape(1, DEC // 2), d3b.reshape(1, 1))
    return out


# speedup vs baseline: 3.8165x; 1.2405x over previous
import jax
import jax.numpy as jnp
from jax import lax
from jax.experimental import pallas as pl
from jax.experimental.pallas import tpu as pltpu
from jax.experimental.pallas import tpu_sc as plsc

N = 10000
E = 320000
P = 320000
D = 128
H = 128
DEC = 256
EPS = 1e-5

NC, NS = 2, 16
NW = NC * NS
CHUNK = 128
CPT = 79
EPAD = NW * CPT * CHUNK
NP = N + 8
ZR = 640
ZLAST = NP - ZR

BP = 512
GD = P // BP

_mesh_cache = []


def _mesh():
    if not _mesh_cache:
        _mesh_cache.append(plsc.VectorSubcoreMesh(
            core_axis_name="c", subcore_axis_name="s",
            num_cores=NC, num_subcores=NS))
    return _mesh_cache[0]


IPT = CPT * CHUNK
VSTEPS = IPT // 16


def _deg_body(ones_hbm, dst_hbm, out_hbm, idx_v, ones_v, acc_sh, sem):
    c = lax.axis_index("c")
    s = lax.axis_index("s")
    wid = s * NC + c
    zoff = jnp.minimum(s * ZR, ZLAST)

    pltpu.sync_copy(ones_hbm.at[pl.ds(0, 16)], ones_v)
    pltpu.sync_copy(ones_hbm.at[pl.ds(zoff, ZR)], acc_sh.at[pl.ds(zoff, ZR)])
    pltpu.sync_copy(dst_hbm.at[wid], idx_v)
    plsc.subcore_barrier()

    def step(t, _):
        dvec = idx_v[0, pl.ds(16 * t, 16)]
        pltpu.sync_copy(ones_v, acc_sh.at[dvec], add=True)
        return 0
    lax.fori_loop(0, VSTEPS, step, 0)

    plsc.subcore_barrier()
    pltpu.sync_copy(acc_sh.at[pl.ds(zoff, ZR)], out_hbm.at[c, pl.ds(zoff, ZR)])


def _sc_deg(ones, dst3d):
    return pl.kernel(
        _deg_body,
        out_type=jax.ShapeDtypeStruct((NC, NP, H), jnp.float32),
        mesh=_mesh(),
        scratch_types=[
            pltpu.VMEM((1, IPT), jnp.int32),
            pltpu.VMEM((16, H), jnp.float32),
            pltpu.VMEM_SHARED((NP, H), jnp.float32),
            pltpu.SemaphoreType.DMA,
        ],
    )(ones, dst3d)


def _conv_body(tbl_hbm, src_hbm, dst_hbm, out_hbm,
               rows0, rows1, sb0, sb1, db0, db1, acc_sh, gs0, gs1):
    c = lax.axis_index("c")
    s = lax.axis_index("s")
    wid = s * NC + c
    zoff = jnp.minimum(s * ZR, ZLAST)
    rows = (rows0, rows1)
    sb = (sb0, sb1)
    db = (db0, db1)
    gs = (gs0, gs1)

    pltpu.sync_copy(tbl_hbm.at[pl.ds(zoff, ZR)], acc_sh.at[pl.ds(zoff, ZR)])
    plsc.subcore_barrier()

    def issue(t, b):
        pltpu.sync_copy(src_hbm.at[wid, 0, pl.ds(t * CHUNK, CHUNK)], sb[b])
        pltpu.sync_copy(dst_hbm.at[wid, 0, pl.ds(t * CHUNK, CHUNK)], db[b])
        pltpu.async_copy(tbl_hbm.at[sb[b]], rows[b], gs[b])

    def drain_scatter(t, b):
        pltpu.make_async_copy(tbl_hbm.at[pl.ds(0, CHUNK)], rows[b],
                              gs[b]).wait()
        pltpu.sync_copy(rows[b], acc_sh.at[db[b]], add=True)

    issue(0, 0)

    def group(g, _):
        t = 2 * g
        drain_scatter(t, 0)
        issue(t + 1, 1)
        drain_scatter(t + 1, 1)
        issue(t + 2, 0)
        return 0
    lax.fori_loop(0, (CPT - 1) // 2, group, 0)
    drain_scatter(CPT - 1, 0)

    plsc.subcore_barrier()
    pltpu.sync_copy(acc_sh.at[pl.ds(zoff, ZR)], out_hbm.at[c, pl.ds(zoff, ZR)])


def _sc_conv(tbl, src3d, dst3d):
    return pl.kernel(
        _conv_body,
        out_type=jax.ShapeDtypeStruct((NC, NP, H), jnp.float32),
        mesh=_mesh(),
        scratch_types=[
            pltpu.VMEM((CHUNK, H), jnp.float32),
            pltpu.VMEM((CHUNK, H), jnp.float32),
            pltpu.VMEM((CHUNK,), jnp.int32),
            pltpu.VMEM((CHUNK,), jnp.int32),
            pltpu.VMEM((CHUNK,), jnp.int32),
            pltpu.VMEM((CHUNK,), jnp.int32),
            pltpu.VMEM_SHARED((NP, H), jnp.float32),
            pltpu.SemaphoreType.DMA,
            pltpu.SemaphoreType.DMA,
        ],
    )(tbl, src3d, dst3d)


def _pair_body(z_hbm, s_hbm, d_hbm, u_hbm, v_hbm,
               ub0, ub1, vb0, vb1, sb0, sb1, db0, db1, gsu0, gsu1, gsv0, gsv1):
    c = lax.axis_index("c")
    s = lax.axis_index("s")
    wid = s * NC + c
    ub = (ub0, ub1)
    vb = (vb0, vb1)
    sb = (sb0, sb1)
    db = (db0, db1)
    gsu = (gsu0, gsu1)
    gsv = (gsv0, gsv1)

    def issue(t, b):
        pltpu.sync_copy(s_hbm.at[wid, 0, pl.ds(t * CHUNK, CHUNK)], sb[b])
        pltpu.sync_copy(d_hbm.at[wid, 0, pl.ds(t * CHUNK, CHUNK)], db[b])
        pltpu.async_copy(z_hbm.at[sb[b]], ub[b], gsu[b])
        pltpu.async_copy(z_hbm.at[db[b]], vb[b], gsv[b])

    def drain_store(t, b):
        base = wid * IPT + t * CHUNK
        pltpu.make_async_copy(z_hbm.at[pl.ds(0, CHUNK)], ub[b], gsu[b]).wait()
        pltpu.make_async_copy(z_hbm.at[pl.ds(0, CHUNK)], vb[b], gsv[b]).wait()
        pltpu.sync_copy(ub[b], u_hbm.at[pl.ds(base, CHUNK)])
        pltpu.sync_copy(vb[b], v_hbm.at[pl.ds(base, CHUNK)])

    issue(0, 0)

    def group(g, _):
        t = 2 * g
        drain_store(t, 0)
        issue(t + 1, 1)
        drain_store(t + 1, 1)
        issue(t + 2, 0)
        return 0
    lax.fori_loop(0, (CPT - 1) // 2, group, 0)
    drain_store(CPT - 1, 0)


def _sc_pair(z, s3d, d3d):
    return pl.kernel(
        _pair_body,
        out_type=(jax.ShapeDtypeStruct((EPAD, H), jnp.float32),
                  jax.ShapeDtypeStruct((EPAD, H), jnp.float32)),
        mesh=_mesh(),
        scratch_types=[
            pltpu.VMEM((CHUNK, H), jnp.float32),
            pltpu.VMEM((CHUNK, H), jnp.float32),
            pltpu.VMEM((CHUNK, H), jnp.float32),
            pltpu.VMEM((CHUNK, H), jnp.float32),
            pltpu.VMEM((CHUNK,), jnp.int32),
            pltpu.VMEM((CHUNK,), jnp.int32),
            pltpu.VMEM((CHUNK,), jnp.int32),
            pltpu.VMEM((CHUNK,), jnp.int32),
            pltpu.SemaphoreType.DMA,
            pltpu.SemaphoreType.DMA,
            pltpu.SemaphoreType.DMA,
            pltpu.SemaphoreType.DMA,
        ],
    )(z, s3d, d3d)


def _prep_body(degp_ref, x_ref, W0_ref, Wp_ref, hs0_ref, xp_ref, dinv_ref):
    deg = degp_ref[0, :N, 0:1] + degp_ref[1, :N, 0:1] - 1.0
    dinv = lax.rsqrt(deg)
    x = x_ref[...]
    hs0_ref[...] = jnp.dot(x, W0_ref[...],
                           preferred_element_type=jnp.float32) * dinv
    xp_ref[...] = jnp.dot(x, Wp_ref[...], preferred_element_type=jnp.float32)
    dinv_ref[...] = dinv


def _tc_prep(degp, x, W0, Wp):
    return pl.pallas_call(
        _prep_body,
        out_shape=(jax.ShapeDtypeStruct((N, H), jnp.float32),
                   jax.ShapeDtypeStruct((N, H), jnp.float32),
                   jax.ShapeDtypeStruct((N, 1), jnp.float32)),
    )(degp, x, W0, Wp)


def _enc1_body(S_ref, hs_ref, dinv_ref, xp_ref, b_ref, g_ref, be_ref, W1_ref,
               h_ref, hs1_ref):
    S = S_ref[0, :N] + S_ref[1, :N] - hs_ref[...]
    dinv = dinv_ref[...]
    pre = S * dinv + b_ref[...]
    mu = jnp.mean(pre, axis=0, keepdims=True)
    var = jnp.mean((pre - mu) ** 2, axis=0, keepdims=True)
    hb = g_ref[...] * (pre - mu) * lax.rsqrt(var + EPS) + be_ref[...]
    h = jnp.maximum(hb, 0.0) + xp_ref[...]
    h_ref[...] = h
    hs1_ref[...] = jnp.dot(h, W1_ref[...],
                           preferred_element_type=jnp.float32) * dinv


def _tc_enc1(S0, hs0, dinv, xp, b0, g0, be0, W1):
    return pl.pallas_call(
        _enc1_body,
        out_shape=(jax.ShapeDtypeStruct((N, H), jnp.float32),
                   jax.ShapeDtypeStruct((N, H), jnp.float32)),
    )(S0, hs0, dinv, xp, b0, g0, be0, W1)


def _enc2_body(S_ref, hs1_ref, dinv_ref, h_ref, b_ref, g_ref, be_ref, z_ref):
    S = S_ref[0, :N] + S_ref[1, :N] - hs1_ref[...]
    pre = S * dinv_ref[...] + b_ref[...]
    mu = jnp.mean(pre, axis=0, keepdims=True)
    var = jnp.mean((pre - mu) ** 2, axis=0, keepdims=True)
    hb = g_ref[...] * (pre - mu) * lax.rsqrt(var + EPS) + be_ref[...]
    z_ref[...] = jnp.maximum(hb, 0.0) + h_ref[...]


def _tc_enc2(S1, hs1, dinv, h, b1, g1, be1):
    return pl.pallas_call(
        _enc2_body,
        out_shape=jax.ShapeDtypeStruct((N, H), jnp.float32),
    )(S1, hs1, dinv, h, b1, g1, be1)


def _dec1_body(u_ref, v_ref, W_ref, b_ref, o_ref, st_ref):
    u = u_ref[...]
    v = v_ref[...]
    o = (jnp.dot(u, W_ref[0], preferred_element_type=jnp.float32)
         + jnp.dot(v, W_ref[1], preferred_element_type=jnp.float32)
         + jnp.dot(u * v, W_ref[2], preferred_element_type=jnp.float32)
         + jnp.dot(jnp.abs(u - v), W_ref[3], preferred_element_type=jnp.float32)
         + b_ref[...])
    o_ref[...] = o
    st = jnp.concatenate(
        [jnp.sum(o, axis=0, keepdims=True),
         jnp.sum(o * o, axis=0, keepdims=True),
         jnp.zeros((6, o.shape[1]), jnp.float32)], axis=0)

    @pl.when(pl.program_id(0) == 0)
    def _():
        st_ref[...] = st

    @pl.when(pl.program_id(0) > 0)
    def _():
        st_ref[...] += st


def _tc_dec1(u, v, W4, b):
    return pl.pallas_call(
        _dec1_body,
        grid=(GD,),
        in_specs=[
            pl.BlockSpec((BP, H), lambda i: (i, 0)),
            pl.BlockSpec((BP, H), lambda i: (i, 0)),
            pl.BlockSpec((4, H, DEC), lambda i: (0, 0, 0)),
            pl.BlockSpec((1, DEC), lambda i: (0, 0)),
        ],
        out_specs=(pl.BlockSpec((BP, DEC), lambda i: (i, 0)),
                   pl.BlockSpec((8, DEC), lambda i: (0, 0))),
        out_shape=(jax.ShapeDtypeStruct((P, DEC), jnp.float32),
                   jax.ShapeDtypeStruct((8, DEC), jnp.float32)),
    )(u, v, W4, b)


def _dec2_body(o1_ref, st_ref, g_ref, be_ref, W_ref, b_ref, o2_ref, st2_ref):
    mu = st_ref[0:1] * (1.0 / P)
    var = st_ref[1:2] * (1.0 / P) - mu * mu
    a = g_ref[...] * lax.rsqrt(var + EPS)
    y = jnp.maximum(a * (o1_ref[...] - mu) + be_ref[...], 0.0)
    o2 = jnp.dot(y, W_ref[...], preferred_element_type=jnp.float32) + b_ref[...]
    o2_ref[...] = o2
    st = jnp.concatenate(
        [jnp.sum(o2, axis=0, keepdims=True),
         jnp.sum(o2 * o2, axis=0, keepdims=True),
         jnp.zeros((6, o2.shape[1]), jnp.float32)], axis=0)

    @pl.when(pl.program_id(0) == 0)
    def _():
        st2_ref[...] = st

    @pl.when(pl.program_id(0) > 0)
    def _():
        st2_ref[...] += st


def _tc_dec2(o1, st1, g, be, W, b):
    return pl.pallas_call(
        _dec2_body,
        grid=(GD,),
        in_specs=[
            pl.BlockSpec((BP, DEC), lambda i: (i, 0)),
            pl.BlockSpec((8, DEC), lambda i: (0, 0)),
            pl.BlockSpec((1, DEC), lambda i: (0, 0)),
            pl.BlockSpec((1, DEC), lambda i: (0, 0)),
            pl.BlockSpec((DEC, DEC // 2), lambda i: (0, 0)),
            pl.BlockSpec((1, DEC // 2), lambda i: (0, 0)),
        ],
        out_specs=(pl.BlockSpec((BP, DEC // 2), lambda i: (i, 0)),
                   pl.BlockSpec((8, DEC // 2), lambda i: (0, 0))),
        out_shape=(jax.ShapeDtypeStruct((P, DEC // 2), jnp.float32),
                   jax.ShapeDtypeStruct((8, DEC // 2), jnp.float32)),
    )(o1, st1, g, be, W, b)


def _dec3_body(o2_ref, st_ref, g_ref, be_ref, w_ref, b_ref, out_ref):
    mu = st_ref[0:1] * (1.0 / P)
    var = st_ref[1:2] * (1.0 / P) - mu * mu
    a = g_ref[...] * lax.rsqrt(var + EPS)
    y = jnp.maximum(a * (o2_ref[...] - mu) + be_ref[...], 0.0)
    out_ref[...] = jnp.sum(y * w_ref[...], axis=1) + b_ref[0, 0]


def _tc_dec3(o2, st2, g, be, w_row, b):
    return pl.pallas_call(
        _dec3_body,
        grid=(GD,),
        in_specs=[
            pl.BlockSpec((BP, DEC // 2), lambda i: (i, 0)),
            pl.BlockSpec((8, DEC // 2), lambda i: (0, 0)),
            pl.BlockSpec((1, DEC // 2), lambda i: (0, 0)),
            pl.BlockSpec((1, DEC // 2), lambda i: (0, 0)),
            pl.BlockSpec((1, DEC // 2), lambda i: (0, 0)),
            pl.BlockSpec((1, 1), lambda i: (0, 0)),
        ],
        out_specs=pl.BlockSpec((BP,), lambda i: (i,)),
        out_shape=jax.ShapeDtypeStruct((P,), jnp.float32),
    )(o2, st2, g, be, w_row, b)


def _pad_idx(a, n_total):
    pad = jnp.full((n_total - a.shape[0],), N, dtype=a.dtype)
    return jnp.concatenate([a, pad]).reshape(NW, 1, CPT * CHUNK)


def _pad_rows(t):
    return jnp.pad(t, ((0, NP - t.shape[0]), (0, 0)))


def kernel(x, edge_index, edge_pairs, W0, b0, g0, be0, W1, b1, g1, be1, Wp,
           d1W, d1b, d1g, d1be, d2W, d2b, d2g, d2be, d3W, d3b):
    src = _pad_idx(edge_index[0], EPAD)
    dst = _pad_idx(edge_index[1], EPAD)
    ps = _pad_idx(edge_pairs[:, 0], EPAD)
    pd = _pad_idx(edge_pairs[:, 1], EPAD)

    degp = _sc_deg(jnp.ones((NP, H), jnp.float32), dst)
    hs0, xp, dinv = _tc_prep(degp, x, W0, Wp)
    S0 = _sc_conv(_pad_rows(hs0), src, dst)
    h, hs1 = _tc_enc1(S0, hs0, dinv, xp, b0.reshape(1, H), g0.reshape(1, H),
                      be0.reshape(1, H), W1)
    S1 = _sc_conv(_pad_rows(hs1), src, dst)
    z = _tc_enc2(S1, hs1, dinv, h, b1.reshape(1, H), g1.reshape(1, H),
                 be1.reshape(1, H))
    u, v = _sc_pair(_pad_rows(z), ps, pd)
    o1, st1 = _tc_dec1(u, v, d1W.reshape(4, H, DEC), d1b.reshape(1, DEC))
    o2, st2 = _tc_dec2(o1, st1, d1g.reshape(1, DEC), d1be.reshape(1, DEC),
                       d2W, d2b.reshape(1, DEC // 2))
    out = _tc_dec3(o2, st2, d2g.reshape(1, DEC // 2), d2be.reshape(1, DEC // 2),
                   d3W.reshape(1, DEC // 2), d3b.reshape(1, 1))
    return out
